# baseline (speedup 1.0000x reference)
import functools

import jax
import jax.numpy as jnp
from jax import lax
from jax.experimental import pallas as pl
from jax.experimental.pallas import tpu as pltpu
from jax.experimental.pallas import tpu_sc as plsc

S = 7.0
NCH = 30
NB = 4096
NW = 32
BPW = NB // NW
NROW = 7
GROUPS = 7 * (BPW // 16)
RING = 2


def _sqrt16(x):
    i = lax.bitcast_convert_type(x, jnp.int32)
    y = lax.bitcast_convert_type(jnp.int32(0x5F3759DF) - (i >> 1), jnp.float32)
    for _ in range(2):
        y = y * (1.5 - 0.5 * x * y * y)
    return x * y


def _xyxy(x, y, w, h):
    cx = x / S
    cy = y / S
    return cx - 0.5 * w, cy - 0.5 * h, cx + 0.5 * w, cy + 0.5 * h


def _group_update(pb, tb, k, acc_no, acc_hv):
    s2 = k // 8
    sl = pl.ds((k % 8) * 16, 16)

    def p(c):
        return pb[s2, c, sl]

    def t(c):
        return tb[s2, c, sl]

    conf = t(4)
    objm = conf > 0.0
    p4, p9 = p(4), p(9)
    acc_no = acc_no + jnp.where(objm, 0.0, p4 * p4 + p9 * p9)

    def heavy_fn(acch):
        t0, t1, t2, t3 = t(0), t(1), t(2), t(3)
        tx1, ty1, tx2, ty2 = _xyxy(t0, t1, t2, t3)
        area2 = (tx2 - tx1) * (ty2 - ty1)

        def iou_parts(x, y, w, h):
            x1, y1, x2, y2 = _xyxy(x, y, w, h)
            iw = jnp.maximum(jnp.minimum(x2, tx2) - jnp.maximum(x1, tx1), 0.0)
            ih = jnp.maximum(jnp.minimum(y2, ty2) - jnp.maximum(y1, ty1), 0.0)
            inter = iw * ih
            area1 = (x2 - x1) * (y2 - y1)
            return inter, area1 + area2 - inter

        p0, p1, p2, p3 = p(0), p(1), p(2), p(3)
        p5, p6, p7, p8 = p(5), p(6), p(7), p(8)
        i0, d0 = iou_parts(p0, p1, p2, p3)
        i1, d1 = iou_parts(p5, p6, p7, p8)
        sel = i1 * d0 > i0 * d1
        maxiou = jnp.where(sel, i1, i0) / jnp.where(sel, d1, d0)

        def pick(a, b):
            return jnp.where(sel, b, a)

        px, py = pick(p0, p5), pick(p1, p6)
        pw, ph = pick(p2, p7), pick(p3, p8)
        pc = pick(p4, p9)
        qx, qy = pick(t0, t(5)), pick(t1, t(6))
        qw, qh = pick(t2, t(7)), pick(t3, t(8))

        dx, dy = px - qx, py - qy
        lxy = dx * dx + dy * dy
        dw = _sqrt16(pw) - _sqrt16(qw)
        dh = _sqrt16(ph) - _sqrt16(qh)
        lwh = dw * dw + dh * dh
        do = pc - maxiou
        lobj = do * do
        lc = [jnp.zeros((16,), jnp.float32) for _ in range(4)]
        for n, c in enumerate(range(10, 30)):
            d = p(c) - t(c)
            lc[n % 4] = lc[n % 4] + d * d
        lcls = (lc[0] + lc[1]) + (lc[2] + lc[3])
        heavy = 5.0 * (lxy + lwh) + lobj + lcls
        return acch + jnp.where(objm, heavy, 0.0)

    acc_hv = heavy_fn(acc_hv)
    return acc_no, acc_hv


def _body(p_hbm, t_hbm, out_hbm,
          pb0, pb1, tb0, tb1, obuf,
          sp0, sp1, st0, st1):
    wid = lax.axis_index("s") * 2 + lax.axis_index("c")
    b0 = wid * BPW
    pbufs = (pb0, pb1)
    tbufs = (tb0, tb1)
    psems = (sp0, sp1)
    tsems = (st0, st1)

    def copies(g, slot):
        cp = pltpu.make_async_copy(
            p_hbm.at[g, :, :, pl.ds(b0, BPW)], pbufs[slot], psems[slot])
        ct = pltpu.make_async_copy(
            t_hbm.at[g, :, :, pl.ds(b0, BPW)], tbufs[slot], tsems[slot])
        return cp, ct

    def start(g, slot):
        cp, ct = copies(g, slot)
        cp.start()
        ct.start()

    def finish(g, slot):
        cp, ct = copies(g, slot)
        cp.wait()
        ct.wait()

    def do_slab(g, slot, acc_no, acc_hv):
        finish(g, slot)

        @plsc.parallel_loop(0, GROUPS, unroll=1, carry=(acc_no, acc_hv))
        def accs(k, carry):
            return _group_update(pbufs[slot], tbufs[slot], k, *carry)

        return accs

    start(0, 0)
    start(1, 1)

    def pair(i, carry):
        acc_no, acc_hv = carry
        g = 2 * i
        acc_no, acc_hv = do_slab(g, 0, acc_no, acc_hv)
        start(g + 2, 0)
        acc_no, acc_hv = do_slab(g + 1, 1, acc_no, acc_hv)

        @pl.when(g + 3 < NROW)
        def _():
            start(g + 3, 1)

        return acc_no, acc_hv

    zero = jnp.zeros((16,), jnp.float32)
    acc_no, acc_hv = lax.fori_loop(0, (NROW - 1) // 2, pair, (zero, zero))
    acc_no, acc_hv = do_slab(NROW - 1, 0, acc_no, acc_hv)

    obuf[...] = (acc_hv + 0.5 * acc_no) * (1.0 / float(NB))
    pltpu.sync_copy(obuf, out_hbm.at[pl.ds(wid * 16, 16)])


@jax.jit
def _sc_loss(p4d, t4d):
    mesh = plsc.VectorSubcoreMesh(core_axis_name="c", subcore_axis_name="s")
    run = functools.partial(
        pl.kernel,
        mesh=mesh,
        compiler_params=pltpu.CompilerParams(use_tc_tiling_on_sc=True),
        out_type=jax.ShapeDtypeStruct((NW * 16,), jnp.float32),
        scratch_types=(
            [pltpu.VMEM((7, NCH, BPW), jnp.float32) for _ in range(4)]
            + [pltpu.VMEM((16,), jnp.float32)]
            + [pltpu.SemaphoreType.DMA for _ in range(4)]
        ),
    )(_body)
    return run(p4d, t4d)


def kernel(pred_tensor, target_tensor):
    p4d = lax.transpose(pred_tensor, (1, 2, 3, 0))
    t4d = lax.transpose(target_tensor, (1, 2, 3, 0))
    parts = _sc_loss(p4d, t4d)
    return jnp.sum(parts)


# speedup vs baseline: 1.0196x; 1.0196x over previous
import functools

import jax
import jax.numpy as jnp
from jax import lax
from jax.experimental import pallas as pl
from jax.experimental.pallas import tpu as pltpu
from jax.experimental.pallas import tpu_sc as plsc

S = 7.0
NCH = 30
NB = 4096
NW = 32
BPW = NB // NW
NROW = 7
GROUPS = 7 * (BPW // 16)
RING = 2


def _sqrt16(x):
    i = lax.bitcast_convert_type(x, jnp.int32)
    y = lax.bitcast_convert_type(jnp.int32(0x5F3759DF) - (i >> 1), jnp.float32)
    for _ in range(2):
        y = y * (1.5 - 0.5 * x * y * y)
    return x * y


def _xyxy(x, y, w, h):
    cx = x / S
    cy = y / S
    return cx - 0.5 * w, cy - 0.5 * h, cx + 0.5 * w, cy + 0.5 * h


def _group_update(pb, tb, k, acc_no, acc_hv):
    s2 = k // 8
    sl = pl.ds((k % 8) * 16, 16)

    def p(c):
        return pb[s2, c, sl]

    def t(c):
        return tb[s2, c, sl]

    conf = t(4)
    objm = conf > 0.0
    p4, p9 = p(4), p(9)
    acc_no = acc_no + jnp.where(objm, 0.0, p4 * p4 + p9 * p9)

    def heavy_fn(acch):
        t0, t1, t2, t3 = t(0), t(1), t(2), t(3)
        tx1, ty1, tx2, ty2 = _xyxy(t0, t1, t2, t3)
        area2 = (tx2 - tx1) * (ty2 - ty1)

        def iou_parts(x, y, w, h):
            x1, y1, x2, y2 = _xyxy(x, y, w, h)
            iw = jnp.maximum(jnp.minimum(x2, tx2) - jnp.maximum(x1, tx1), 0.0)
            ih = jnp.maximum(jnp.minimum(y2, ty2) - jnp.maximum(y1, ty1), 0.0)
            inter = iw * ih
            area1 = (x2 - x1) * (y2 - y1)
            return inter, area1 + area2 - inter

        p0, p1, p2, p3 = p(0), p(1), p(2), p(3)
        p5, p6, p7, p8 = p(5), p(6), p(7), p(8)
        i0, d0 = iou_parts(p0, p1, p2, p3)
        i1, d1 = iou_parts(p5, p6, p7, p8)
        sel = i1 * d0 > i0 * d1
        maxiou = jnp.where(sel, i1, i0) / jnp.where(sel, d1, d0)

        def pick(a, b):
            return jnp.where(sel, b, a)

        px, py = pick(p0, p5), pick(p1, p6)
        pw, ph = pick(p2, p7), pick(p3, p8)
        pc = pick(p4, p9)
        qx, qy = pick(t0, t(5)), pick(t1, t(6))
        qw, qh = pick(t2, t(7)), pick(t3, t(8))

        dx, dy = px - qx, py - qy
        lxy = dx * dx + dy * dy
        lwh = (pw + qw - 2.0 * _sqrt16(pw * qw)
               + ph + qh - 2.0 * _sqrt16(ph * qh))
        do = pc - maxiou
        lobj = do * do
        lcls = jnp.zeros((16,), jnp.float32)
        for c in range(10, 30):
            d = p(c) - t(c)
            lcls = lcls + d * d
        heavy = 5.0 * (lxy + lwh) + lobj + lcls
        return acch + jnp.where(objm, heavy, 0.0)

    acc_hv = heavy_fn(acc_hv)
    return acc_no, acc_hv


def _body(p_hbm, t_hbm, out_hbm,
          pb0, pb1, tb0, tb1, obuf,
          sp0, sp1, st0, st1):
    wid = lax.axis_index("s") * 2 + lax.axis_index("c")
    b0 = wid * BPW
    pbufs = (pb0, pb1)
    tbufs = (tb0, tb1)
    psems = (sp0, sp1)
    tsems = (st0, st1)

    def copies(g, slot):
        cp = pltpu.make_async_copy(
            p_hbm.at[g, :, :, pl.ds(b0, BPW)], pbufs[slot], psems[slot])
        ct = pltpu.make_async_copy(
            t_hbm.at[g, :, :, pl.ds(b0, BPW)], tbufs[slot], tsems[slot])
        return cp, ct

    def start(g, slot):
        cp, ct = copies(g, slot)
        cp.start()
        ct.start()

    def finish(g, slot):
        cp, ct = copies(g, slot)
        cp.wait()
        ct.wait()

    def do_slab(g, slot, acc_no, acc_hv):
        finish(g, slot)

        @plsc.parallel_loop(0, GROUPS, unroll=1, carry=(acc_no, acc_hv))
        def accs(k, carry):
            return _group_update(pbufs[slot], tbufs[slot], k, *carry)

        return accs

    start(0, 0)
    start(1, 1)

    def pair(i, carry):
        acc_no, acc_hv = carry
        g = 2 * i
        acc_no, acc_hv = do_slab(g, 0, acc_no, acc_hv)
        start(g + 2, 0)
        acc_no, acc_hv = do_slab(g + 1, 1, acc_no, acc_hv)

        @pl.when(g + 3 < NROW)
        def _():
            start(g + 3, 1)

        return acc_no, acc_hv

    zero = jnp.zeros((16,), jnp.float32)
    acc_no, acc_hv = lax.fori_loop(0, (NROW - 1) // 2, pair, (zero, zero))
    acc_no, acc_hv = do_slab(NROW - 1, 0, acc_no, acc_hv)

    obuf[...] = (acc_hv + 0.5 * acc_no) * (1.0 / float(NB))
    pltpu.sync_copy(obuf, out_hbm.at[pl.ds(wid * 16, 16)])


@jax.jit
def _sc_loss(p4d, t4d):
    mesh = plsc.VectorSubcoreMesh(core_axis_name="c", subcore_axis_name="s")
    run = functools.partial(
        pl.kernel,
        mesh=mesh,
        compiler_params=pltpu.CompilerParams(use_tc_tiling_on_sc=True),
        out_type=jax.ShapeDtypeStruct((NW * 16,), jnp.float32),
        scratch_types=(
            [pltpu.VMEM((7, NCH, BPW), jnp.float32) for _ in range(4)]
            + [pltpu.VMEM((16,), jnp.float32)]
            + [pltpu.SemaphoreType.DMA for _ in range(4)]
        ),
    )(_body)
    return run(p4d, t4d)


def kernel(pred_tensor, target_tensor):
    p4d = lax.transpose(pred_tensor, (1, 2, 3, 0))
    t4d = lax.transpose(target_tensor, (1, 2, 3, 0))
    parts = _sc_loss(p4d, t4d)
    return jnp.sum(parts)
